# baseline (speedup 1.0000x reference)
import jax
import jax.numpy as jnp
from jax.experimental import pallas as pl
from jax.experimental.pallas import tpu as pltpu

SIZE = 1024
SMOOTHING = 0.1
CONFIDENCE = 1.0 - SMOOTHING
T = 16384

ROWS = 512
LOG19 = 0.6418538861723947


def _loss_kernel(x_ref, tgt_ref, mask_ref, out_ref):
    i = pl.program_id(0)
    x = x_ref[...]
    m = mask_ref[...] != 0
    t = tgt_ref[0, 0, :]

    mf = m.astype(jnp.float32)
    cnt = jnp.float32(SIZE) - jnp.sum(mf, axis=1)
    rowsum = jnp.sum(x, axis=1)
    sv = jnp.sum(jnp.where(m, x, 0.0), axis=1)
    su = rowsum - sv

    col = jax.lax.broadcasted_iota(jnp.int32, (ROWS, SIZE), 1)
    onehot = col == t[:, None]
    x_t = jnp.sum(jnp.where(onehot, x, 0.0), axis=1)
    v_t = jnp.sum(jnp.where(onehot, mf, 0.0), axis=1)

    has_unv = cnt > 0.0
    s = SMOOTHING / jnp.maximum(cnt, 1.0)
    log_s = jnp.log(s)
    base = -sv + jnp.where(has_unv, SMOOTHING * log_s - s * su, 0.0)

    corr_vis = jnp.float32(1.9 * LOG19) - 0.9 * x_t
    sp = s + CONFIDENCE
    corr_unv = sp * jnp.log(sp) - s * log_s - 0.9 * x_t
    corr = jnp.where(v_t > 0.5, corr_vis, corr_unv)

    block_loss = jnp.sum(base + corr).reshape(1, 1)

    @pl.when(i == 0)
    def _init():
        out_ref[...] = jnp.zeros((1, 1), jnp.float32)

    out_ref[...] += block_loss


@jax.jit
def kernel(x, target, visited_mask):
    nblk = T // ROWS
    tgt3 = target.reshape(nblk, 1, ROWS)
    out = pl.pallas_call(
        _loss_kernel,
        grid=(nblk,),
        in_specs=[
            pl.BlockSpec((ROWS, SIZE), lambda i: (i, 0)),
            pl.BlockSpec((1, 1, ROWS), lambda i: (i, 0, 0)),
            pl.BlockSpec((ROWS, SIZE), lambda i: (i, 0)),
        ],
        out_specs=pl.BlockSpec((1, 1), lambda i: (0, 0)),
        out_shape=jax.ShapeDtypeStruct((1, 1), jnp.float32),
    )(x, tgt3, visited_mask.view(jnp.int8))
    return out[0, 0]


# speedup vs baseline: 1.2922x; 1.2922x over previous
import jax
import jax.numpy as jnp
from jax.experimental import pallas as pl
from jax.experimental.pallas import tpu as pltpu

SIZE = 1024
SMOOTHING = 0.1
CONFIDENCE = 1.0 - SMOOTHING
T = 16384

ROWS = 1024
LOG19 = 0.6418538861723947


def _loss_kernel(x_ref, tgt_ref, mask_ref, out_ref):
    i = pl.program_id(0)
    x = x_ref[...]
    t = tgt_ref[0, 0, :]

    mf = mask_ref[...].astype(jnp.float32)
    cnt = jnp.float32(SIZE) - jnp.sum(mf, axis=1)
    rowsum = jnp.sum(x, axis=1)
    sv = jnp.sum(x * mf, axis=1)
    su = rowsum - sv

    col = jax.lax.broadcasted_iota(jnp.int32, (ROWS, SIZE), 1)
    onehot = col == t[:, None]
    x_t = jnp.sum(jnp.where(onehot, x, 0.0), axis=1)
    v_t = jnp.sum(jnp.where(onehot, mf, 0.0), axis=1)

    has_unv = cnt > 0.0
    s = SMOOTHING / jnp.maximum(cnt, 1.0)
    log_s = jnp.log(s)
    base = -sv + jnp.where(has_unv, SMOOTHING * log_s - s * su, 0.0)

    corr_vis = jnp.float32(1.9 * LOG19) - 0.9 * x_t
    sp = s + CONFIDENCE
    corr_unv = sp * jnp.log(sp) - s * log_s - 0.9 * x_t
    corr = jnp.where(v_t > 0.5, corr_vis, corr_unv)

    block_loss = jnp.sum(base + corr).reshape(1, 1)

    @pl.when(i == 0)
    def _init():
        out_ref[...] = jnp.zeros((1, 1), jnp.float32)

    out_ref[...] += block_loss


@jax.jit
def kernel(x, target, visited_mask):
    nblk = T // ROWS
    tgt3 = target.reshape(nblk, 1, ROWS)
    out = pl.pallas_call(
        _loss_kernel,
        grid=(nblk,),
        in_specs=[
            pl.BlockSpec((ROWS, SIZE), lambda i: (i, 0)),
            pl.BlockSpec((1, 1, ROWS), lambda i: (i, 0, 0)),
            pl.BlockSpec((ROWS, SIZE), lambda i: (i, 0)),
        ],
        out_specs=pl.BlockSpec((1, 1), lambda i: (0, 0)),
        out_shape=jax.ShapeDtypeStruct((1, 1), jnp.float32),
    )(x, tgt3, visited_mask.view(jnp.int8))
    return out[0, 0]


# speedup vs baseline: 1.3178x; 1.0198x over previous
import jax
import jax.numpy as jnp
from jax.experimental import pallas as pl
from jax.experimental.pallas import tpu as pltpu

SIZE = 1024
SMOOTHING = 0.1
CONFIDENCE = 1.0 - SMOOTHING
T = 16384

ROWS = 2048
LOG19 = 0.6418538861723947


def _loss_kernel(x_ref, tgt_ref, mask_ref, out_ref):
    i = pl.program_id(0)
    x = x_ref[...]
    t = tgt_ref[0, 0, :]

    mf = mask_ref[...].astype(jnp.float32)
    cnt = jnp.float32(SIZE) - jnp.sum(mf, axis=1)
    rowsum = jnp.sum(x, axis=1)
    sv = jnp.sum(x * mf, axis=1)
    su = rowsum - sv

    col = jax.lax.broadcasted_iota(jnp.int32, (ROWS, SIZE), 1)
    onehot = col == t[:, None]
    x_t = jnp.sum(jnp.where(onehot, x, 0.0), axis=1)
    v_t = jnp.sum(jnp.where(onehot, mf, 0.0), axis=1)

    has_unv = cnt > 0.0
    s = SMOOTHING / jnp.maximum(cnt, 1.0)
    log_s = jnp.log(s)
    base = -sv + jnp.where(has_unv, SMOOTHING * log_s - s * su, 0.0)

    corr_vis = jnp.float32(1.9 * LOG19) - 0.9 * x_t
    sp = s + CONFIDENCE
    corr_unv = sp * jnp.log(sp) - s * log_s - 0.9 * x_t
    corr = jnp.where(v_t > 0.5, corr_vis, corr_unv)

    block_loss = jnp.sum(base + corr).reshape(1, 1)

    @pl.when(i == 0)
    def _init():
        out_ref[...] = jnp.zeros((1, 1), jnp.float32)

    out_ref[...] += block_loss


@jax.jit
def kernel(x, target, visited_mask):
    nblk = T // ROWS
    tgt3 = target.reshape(nblk, 1, ROWS)
    out = pl.pallas_call(
        _loss_kernel,
        grid=(nblk,),
        in_specs=[
            pl.BlockSpec((ROWS, SIZE), lambda i: (i, 0)),
            pl.BlockSpec((1, 1, ROWS), lambda i: (i, 0, 0)),
            pl.BlockSpec((ROWS, SIZE), lambda i: (i, 0)),
        ],
        out_specs=pl.BlockSpec((1, 1), lambda i: (0, 0)),
        out_shape=jax.ShapeDtypeStruct((1, 1), jnp.float32),
    )(x, tgt3, visited_mask.view(jnp.int8))
    return out[0, 0]


# speedup vs baseline: 1.3199x; 1.0016x over previous
import jax
import jax.numpy as jnp
from jax.experimental import pallas as pl

SIZE = 1024
SMOOTHING = 0.1
CONFIDENCE = 1.0 - SMOOTHING
T = 16384

ROWS = 2048
LOG19 = 0.6418538861723947


def _loss_kernel(x_ref, tgt_ref, mask_ref, out_ref):
    i = pl.program_id(0)
    x = x_ref[...]
    t = tgt_ref[0, 0, :]

    mf = mask_ref[...].astype(jnp.float32)
    cnt = jnp.float32(SIZE) - jnp.sum(mf, axis=1)
    rowsum = jnp.sum(x, axis=1)
    sv = jnp.sum(x * mf, axis=1)
    su = rowsum - sv

    col = jax.lax.broadcasted_iota(jnp.int32, (ROWS, SIZE), 1)
    onehot = col == t[:, None]
    x_t = jnp.sum(jnp.where(onehot, x, 0.0), axis=1)
    v_t = jnp.sum(jnp.where(onehot, mf, 0.0), axis=1)

    has_unv = cnt > 0.0
    s = SMOOTHING / jnp.maximum(cnt, 1.0)
    log_s = jnp.log(s)
    base = -sv + jnp.where(has_unv, SMOOTHING * log_s - s * su, 0.0)

    corr_vis = jnp.float32(1.9 * LOG19) - 0.9 * x_t
    sp = s + CONFIDENCE
    corr_unv = sp * jnp.log(sp) - s * log_s - 0.9 * x_t
    corr = jnp.where(v_t > 0.5, corr_vis, corr_unv)

    block_loss = jnp.sum(base + corr).reshape(1, 1)

    @pl.when(i == 0)
    def _init():
        out_ref[...] = jnp.zeros((1, 1), jnp.float32)

    out_ref[...] += block_loss


@jax.jit
def kernel(x, target, visited_mask):
    nblk = T // ROWS
    tgt3 = target.reshape(nblk, 1, ROWS)
    out = pl.pallas_call(
        _loss_kernel,
        grid=(nblk,),
        in_specs=[
            pl.BlockSpec((ROWS, SIZE), lambda i: (i, 0)),
            pl.BlockSpec((1, 1, ROWS), lambda i: (i, 0, 0)),
            pl.BlockSpec((ROWS, SIZE), lambda i: (i, 0)),
        ],
        out_specs=pl.BlockSpec((1, 1), lambda i: (0, 0)),
        out_shape=jax.ShapeDtypeStruct((1, 1), jnp.float32),
    )(x, tgt3, visited_mask.view(jnp.int8))
    return out[0, 0]
